# Initial kernel scaffold; baseline (speedup 1.0000x reference)
import jax
import jax.numpy as jnp
from jax.experimental import pallas as pl


def kernel(pos_embed, token_type_ids):
    raise NotImplementedError("write your pallas kernel here")


# speedup vs baseline: 1.2079x; 1.2079x over previous
import functools

import numpy as np
import jax
import jax.numpy as jnp
from jax import lax
from jax.experimental import pallas as pl
from jax.experimental.pallas import tpu as pltpu
from jax.experimental.pallas import tpu_sc as plsc

_SEQ_LEN = 4096
_D_MODEL = 1024
_NUM_BLOCKS = 4
_CLS_ID = 2


def _rel_indices(seq_len: int, num_blocks: int) -> np.ndarray:
    zero_offset = seq_len * 2
    pos = np.arange(seq_len)
    idx_list = []
    for b in range(num_blocks):
        if b > 0:
            cls_pos = np.array([-(2 ** b) + 1])
            pooled = np.concatenate([cls_pos, pos[1:-1][::2]])
            stride = 2 ** (b - 1)
            ref_point = pooled[0] - pos[0]
            num_remove = 2 * len(pooled)
            max_dist = ref_point + num_remove * stride
            min_dist = pooled[0] - pos[-1]
            idx_list.append(np.arange(max_dist, min_dist - 1, -stride) + zero_offset)
            pos = pooled
        stride = 2 ** b
        max_dist = len(pos) * stride
        min_dist = pos[0] - pos[-1]
        idx_list.append(np.arange(max_dist, min_dist - 1, -stride) + zero_offset)
    return np.concatenate(idx_list).astype(np.int32)


_IDX = _rel_indices(_SEQ_LEN, _NUM_BLOCKS)
_NROWS = _IDX.shape[0]
_NW = 32
_BPW = _NROWS // _NW
_CH = 32
_NCH = _BPW // _CH


def _sc_gather(table, idx):
    mesh = plsc.VectorSubcoreMesh(core_axis_name="c", subcore_axis_name="s")

    @functools.partial(
        pl.kernel,
        mesh=mesh,
        out_type=jax.ShapeDtypeStruct((_NROWS, _D_MODEL), jnp.float32),
        scratch_types=[
            pltpu.VMEM((_BPW,), jnp.int32),
            pltpu.VMEM((_CH, _D_MODEL), jnp.float32),
            pltpu.SemaphoreType.DMA,
        ],
    )
    def k(table_hbm, idx_hbm, out_hbm, idx_v, buf_v, gsem):
        wid = lax.axis_index("s") * 2 + lax.axis_index("c")
        base = pl.multiple_of(wid * _BPW, 8)
        pltpu.sync_copy(idx_hbm.at[pl.ds(base, _BPW)], idx_v)

        def body(j, carry):
            src = table_hbm.at[idx_v.at[pl.ds(j * _CH, _CH)]]
            pltpu.async_copy(src, buf_v, gsem).wait()
            off = pl.multiple_of(base + j * _CH, 8)
            pltpu.sync_copy(buf_v, out_hbm.at[pl.ds(off, _CH)])
            return carry

        lax.fori_loop(0, _NCH, body, 0)

    return k(table, idx)


_BI = 512
_NI = _SEQ_LEN // _BI


def _tc_body(ids_row_ref, ids_col_ref, mat_ref, cls_ref):
    i = pl.program_id(0)
    b = pl.program_id(1)
    row = ids_row_ref[0]
    col = ids_col_ref[0]
    mat_ref[0] = (col == row) | (col == _CLS_ID) | (row == _CLS_ID)

    @pl.when(b == 0)
    def _():
        r = lax.broadcasted_iota(jnp.int32, (_BI, _SEQ_LEN), 0) + i * _BI
        c = lax.broadcasted_iota(jnp.int32, (_BI, _SEQ_LEN), 1)
        cls_ref[...] = ((r > 0) & (c > 0)).astype(jnp.float32)


def _tc_call(tti):
    nb = tti.shape[0]
    ids_row = tti.reshape(nb, 1, _SEQ_LEN)
    ids_col = tti.reshape(nb, _SEQ_LEN, 1)
    return pl.pallas_call(
        _tc_body,
        grid=(_NI, nb),
        in_specs=[
            pl.BlockSpec((1, 1, _SEQ_LEN), lambda i, b: (b, 0, 0)),
            pl.BlockSpec((1, _BI, 1), lambda i, b: (b, i, 0)),
        ],
        out_specs=[
            pl.BlockSpec((1, _BI, _SEQ_LEN), lambda i, b: (b, i, 0)),
            pl.BlockSpec((_BI, _SEQ_LEN), lambda i, b: (i, 0)),
        ],
        out_shape=[
            jax.ShapeDtypeStruct((nb, _SEQ_LEN, _SEQ_LEN), jnp.bool_),
            jax.ShapeDtypeStruct((_SEQ_LEN, _SEQ_LEN), jnp.float32),
        ],
    )(ids_row, ids_col)


def kernel(pos_embed, token_type_ids):
    tti = token_type_ids.astype(jnp.int32)
    idx = jnp.asarray(_IDX)
    pos_out = _sc_gather(pos_embed, idx)
    token_type_mat, cls_mask = _tc_call(tti)
    return (pos_out, token_type_mat, cls_mask)
